# Initial kernel scaffold; baseline (speedup 1.0000x reference)
import jax
import jax.numpy as jnp
from jax.experimental import pallas as pl


def kernel(pred, label):
    raise NotImplementedError("write your pallas kernel here")


# speedup vs baseline: 2.4179x; 2.4179x over previous
import functools

import jax
import jax.numpy as jnp
from jax import lax
from jax.experimental import pallas as pl
from jax.experimental.pallas import tpu as pltpu
from jax.experimental.pallas import tpu_sc as plsc

_N_CELLS = 16384 * 7 * 7
_N_CH = 12
_LANES = 16
_CHUNK = 128
_SENTINEL = _N_CELLS

_mesh = plsc.VectorSubcoreMesh(core_axis_name="c", subcore_axis_name="s")


@functools.partial(
    pl.kernel,
    mesh=_mesh,
    out_type=jax.ShapeDtypeStruct((_LANES,), jnp.float32),
    scratch_types=[
        pltpu.VMEM((_CHUNK,), jnp.int32),
        pltpu.VMEM((_CHUNK,), jnp.float32),
        pltpu.VMEM((_LANES,), jnp.float32),
        pltpu.SMEM((1,), jnp.int32),
        pltpu.SemaphoreType.DMA,
    ],
)
def _first_noobj_row(label_hbm, out_hbm, idx_v, conf_v, row_v, found_ref, sem):
    cid = lax.axis_index("c")
    sid = lax.axis_index("s")
    is_leader = jnp.logical_and(cid == 0, sid == 0)
    lane = lax.broadcasted_iota(jnp.int32, (_LANES,), 0)

    def scan_chunk(start):
        for j in range(_CHUNK // _LANES):
            cell = start + (j * _LANES + lane)
            idx_v[pl.ds(j * _LANES, _LANES)] = cell * _N_CH + 4
        pltpu.async_copy(label_hbm.at[idx_v], conf_v, sem).wait()
        found_v = jnp.full((_LANES,), _SENTINEL, jnp.int32)
        for j in range(_CHUNK // _LANES):
            v = conf_v[pl.ds(j * _LANES, _LANES)]
            cand = jnp.where(v == 0.0, start + (j * _LANES + lane),
                             jnp.int32(_SENTINEL))
            found_v = jnp.minimum(found_v, cand)
        found = jnp.int32(_SENTINEL)
        for l in range(_LANES):
            found = jnp.minimum(found, found_v[l])
        return found

    @pl.when(is_leader)
    def _():
        found_ref[0] = scan_chunk(jnp.int32(0))

        @pl.when(found_ref[0] >= _SENTINEL)
        def _():
            def body(i, c):
                @pl.when(found_ref[0] >= _SENTINEL)
                def _():
                    f = scan_chunk((i + 1) * _CHUNK)

                    @pl.when(f < _SENTINEL)
                    def _():
                        found_ref[0] = f

                return c

            lax.fori_loop(0, _N_CELLS // _CHUNK - 1, body, jnp.int32(0))

        row = jnp.where(found_ref[0] >= _SENTINEL, jnp.int32(0), found_ref[0])
        gidx = row * _N_CH + jnp.minimum(lane, _N_CH - 1)
        pltpu.async_copy(label_hbm.at[gidx], row_v, sem).wait()
        pltpu.sync_copy(row_v, out_hbm)


def kernel(pred, label):
    del pred
    out16 = _first_noobj_row(label.reshape(-1))
    return out16[:_N_CH]


# speedup vs baseline: 40.3711x; 16.6971x over previous
import functools

import jax
import jax.numpy as jnp
from jax import lax
from jax.experimental import pallas as pl
from jax.experimental.pallas import tpu as pltpu
from jax.experimental.pallas import tpu_sc as plsc

_B = 16384
_S = 7
_N_CH = 12
_CELLS_PER_B = _S * _S
_N_CELLS = _B * _CELLS_PER_B
_LANES = 16
_W = 128
_N_ROUNDS = _B // _W
_SENTINEL = _N_CELLS

_mesh = plsc.VectorSubcoreMesh(core_axis_name="c", subcore_axis_name="s")


@functools.partial(
    pl.kernel,
    mesh=_mesh,
    out_type=jax.ShapeDtypeStruct((_LANES,), jnp.float32),
    scratch_types=[
        pltpu.VMEM((_S, _S, _W), jnp.float32),
        pltpu.VMEM((_N_CH, _LANES), jnp.float32),
        pltpu.VMEM((_LANES,), jnp.float32),
        pltpu.SMEM((1,), jnp.int32),
        pltpu.SemaphoreType.DMA,
    ],
    compiler_params=pltpu.CompilerParams(use_tc_tiling_on_sc=True),
)
def _first_noobj_row(xt_hbm, out_hbm, conf_v, row_v, stage_v, found_ref, sem):
    cid = lax.axis_index("c")
    sid = lax.axis_index("s")
    is_leader = jnp.logical_and(cid == 0, sid == 0)
    lane = lax.broadcasted_iota(jnp.int32, (_LANES,), 0)

    def scan_round(b0):
        copies = [
            pltpu.async_copy(
                xt_hbm.at[i * _N_CH + 4, :, pl.ds(b0, _W)], conf_v.at[i], sem)
            for i in range(_S)
        ]
        for c in copies:
            c.wait()
        found_v = jnp.full((_LANES,), _SENTINEL, jnp.int32)
        for i in range(_S):
            for j in range(_S):
                ij = i * _S + j
                for k in range(_W // _LANES):
                    v = conf_v[i, j, pl.ds(k * _LANES, _LANES)]
                    cell = (b0 + k * _LANES + lane) * _CELLS_PER_B + ij
                    found_v = jnp.minimum(
                        found_v,
                        jnp.where(v == 0.0, cell, jnp.int32(_SENTINEL)))
        found = jnp.int32(_SENTINEL)
        for l in range(_LANES):
            found = jnp.minimum(found, found_v[l])
        return found

    @pl.when(is_leader)
    def _():
        found_ref[0] = scan_round(jnp.int32(0))

        @pl.when(found_ref[0] >= _SENTINEL)
        def _():
            def body(r, c):
                @pl.when(found_ref[0] >= _SENTINEL)
                def _():
                    f = scan_round((r + 1) * _W)

                    @pl.when(f < _SENTINEL)
                    def _():
                        found_ref[0] = f

                return c

            lax.fori_loop(0, _N_ROUNDS - 1, body, jnp.int32(0))

        cell = jnp.where(found_ref[0] >= _SENTINEL, jnp.int32(0), found_ref[0])
        b = cell // _CELLS_PER_B
        ij = cell % _CELLS_PER_B
        i = ij // _S
        j = ij % _S
        fetches = [
            pltpu.async_copy(
                xt_hbm.at[i * _N_CH + c, j, pl.ds(b, 1)],
                row_v.at[c, pl.ds(0, 1)], sem)
            for c in range(_N_CH)
        ]
        for f in fetches:
            f.wait()
        out_vec = jnp.zeros((_LANES,), jnp.float32)
        for c in range(_N_CH):
            word = row_v[c]
            out_vec = jnp.where(lane == c, word[0], out_vec)
        stage_v[...] = out_vec
        pltpu.sync_copy(stage_v, out_hbm)


def kernel(pred, label):
    del pred
    xt = jnp.transpose(label, (1, 3, 2, 0)).reshape(_S * _N_CH, _S, _B)
    out16 = _first_noobj_row(xt)
    return out16[:_N_CH]


# speedup vs baseline: 44.2969x; 1.0972x over previous
import functools

import jax
import jax.numpy as jnp
from jax import lax
from jax.experimental import pallas as pl
from jax.experimental.pallas import tpu as pltpu
from jax.experimental.pallas import tpu_sc as plsc

_B = 16384
_S = 7
_N_CH = 12
_CELLS_PER_B = _S * _S
_N_CELLS = _B * _CELLS_PER_B
_LANES = 16
_W = 128
_N_ROUNDS = _B // _W
_SENTINEL = _N_CELLS

_mesh = plsc.VectorSubcoreMesh(
    core_axis_name="c", subcore_axis_name="s", num_cores=1)


@functools.partial(
    pl.kernel,
    mesh=_mesh,
    out_type=jax.ShapeDtypeStruct((_LANES,), jnp.float32),
    scratch_types=[
        pltpu.VMEM((_S, _S, _W), jnp.float32),
        pltpu.VMEM((_N_CH, _LANES), jnp.float32),
        pltpu.VMEM((_LANES,), jnp.float32),
        pltpu.SMEM((1,), jnp.int32),
        pltpu.SemaphoreType.DMA,
    ],
    compiler_params=pltpu.CompilerParams(use_tc_tiling_on_sc=True),
)
def _first_noobj_row(xt_hbm, out_hbm, conf_v, row_v, stage_v, found_ref, sem):
    cid = lax.axis_index("c")
    sid = lax.axis_index("s")
    is_leader = jnp.logical_and(cid == 0, sid == 0)
    lane = lax.broadcasted_iota(jnp.int32, (_LANES,), 0)

    def scan_round(b0, k_lo=0):
        copies = [
            pltpu.async_copy(
                xt_hbm.at[i * _N_CH + 4, :, pl.ds(b0, _W)], conf_v.at[i], sem)
            for i in range(_S)
        ]
        for c in copies:
            c.wait()
        found_v = jnp.full((_LANES,), _SENTINEL, jnp.int32)
        for i in range(_S):
            for j in range(_S):
                ij = i * _S + j
                for k in range(k_lo, _W // _LANES):
                    v = conf_v[i, j, pl.ds(k * _LANES, _LANES)]
                    cell = (b0 + k * _LANES + lane) * _CELLS_PER_B + ij
                    found_v = jnp.minimum(
                        found_v,
                        jnp.where(v == 0.0, cell, jnp.int32(_SENTINEL)))
        found = jnp.int32(_SENTINEL)
        for l in range(_LANES):
            found = jnp.minimum(found, found_v[l])
        return found

    def scan_prefix():
        copies = [
            pltpu.async_copy(
                xt_hbm.at[i * _N_CH + 4, :, pl.ds(0, _W)], conf_v.at[i], sem)
            for i in range(_S)
        ]
        for c in copies:
            c.wait()
        found_v = jnp.full((_LANES,), _SENTINEL, jnp.int32)
        for i in range(_S):
            for j in range(_S):
                ij = i * _S + j
                v = conf_v[i, j, pl.ds(0, _LANES)]
                cell = lane * _CELLS_PER_B + ij
                found_v = jnp.minimum(
                    found_v, jnp.where(v == 0.0, cell, jnp.int32(_SENTINEL)))
        found = jnp.int32(_SENTINEL)
        for l in range(_LANES):
            found = jnp.minimum(found, found_v[l])
        return found

    @pl.when(is_leader)
    def _():
        found_ref[0] = scan_prefix()

        @pl.when(found_ref[0] >= _SENTINEL)
        def _():
            f0 = scan_round(jnp.int32(0), k_lo=1)

            @pl.when(f0 < _SENTINEL)
            def _():
                found_ref[0] = f0

        @pl.when(found_ref[0] >= _SENTINEL)
        def _():
            def body(r, c):
                @pl.when(found_ref[0] >= _SENTINEL)
                def _():
                    f = scan_round((r + 1) * _W)

                    @pl.when(f < _SENTINEL)
                    def _():
                        found_ref[0] = f

                return c

            lax.fori_loop(0, _N_ROUNDS - 1, body, jnp.int32(0))

        cell = jnp.where(found_ref[0] >= _SENTINEL, jnp.int32(0), found_ref[0])
        b = cell // _CELLS_PER_B
        ij = cell % _CELLS_PER_B
        i = ij // _S
        j = ij % _S
        fetches = [
            pltpu.async_copy(
                xt_hbm.at[i * _N_CH + c, j, pl.ds(b, 1)],
                row_v.at[c, pl.ds(0, 1)], sem)
            for c in range(_N_CH)
        ]
        for f in fetches:
            f.wait()
        out_vec = jnp.zeros((_LANES,), jnp.float32)
        for c in range(_N_CH):
            word = row_v[c]
            out_vec = jnp.where(lane == c, word[0], out_vec)
        stage_v[...] = out_vec
        pltpu.sync_copy(stage_v, out_hbm)


def kernel(pred, label):
    del pred
    xt = jnp.transpose(label, (1, 3, 2, 0)).reshape(_S * _N_CH, _S, _B)
    out16 = _first_noobj_row(xt)
    return out16[:_N_CH]
